# baseline (speedup 1.0000x reference)
import functools

import jax
import jax.numpy as jnp
from jax import lax
from jax.experimental import pallas as pl
from jax.experimental.pallas import tpu as pltpu, tpu_sc as plsc

_CHUNK_ROWS = 512


def kernel(x, emb):
    n = x.shape[1]
    d = emb.shape[1]
    info = plsc.get_sparse_core_info()
    nc = info.num_cores
    rows_c = n // nc
    n_ch = rows_c // _CHUNK_ROWS
    mesh = plsc.ScalarSubcoreMesh(axis_name="c", num_cores=nc)

    @functools.partial(
        pl.kernel,
        mesh=mesh,
        out_type=jax.ShapeDtypeStruct((n, d), emb.dtype),
        scratch_types=[
            pltpu.VMEM_SHARED((2, _CHUNK_ROWS, d), emb.dtype),
            pltpu.SemaphoreType.DMA((2,)),
            pltpu.SemaphoreType.DMA((2,)),
        ],
    )
    def run(emb_hbm, out_hbm, buf, isem, osem):
        base = lax.axis_index("c") * rows_c

        def in_copy(i, b):
            return pltpu.make_async_copy(
                emb_hbm.at[pl.ds(base + i * _CHUNK_ROWS, _CHUNK_ROWS), :],
                buf.at[b],
                isem.at[b],
            )

        def out_copy(i, b):
            return pltpu.make_async_copy(
                buf.at[b],
                out_hbm.at[pl.ds(base + i * _CHUNK_ROWS, _CHUNK_ROWS), :],
                osem.at[b],
            )

        in_copy(0, 0).start()
        for i in range(n_ch):
            b = i % 2
            in_copy(i, b).wait()
            out_copy(i, b).start()
            if i + 1 < n_ch:
                nb = (i + 1) % 2
                if i >= 1:
                    out_copy(i - 1, nb).wait()
                in_copy(i + 1, nb).start()
        for i in range(max(0, n_ch - 2), n_ch):
            out_copy(i, i % 2).wait()

    return run(emb)


# speedup vs baseline: 3.0184x; 3.0184x over previous
import functools

import jax
import jax.numpy as jnp
from jax.experimental import pallas as pl
from jax.experimental.pallas import tpu as pltpu

_CHUNK = 512


def _copy_kernel(n_rows, d, emb_ref, out_ref, bufs, in_sems, out_sems):
    n_chunks = n_rows // _CHUNK

    def in_copy(i):
        return pltpu.make_async_copy(
            emb_ref.at[pl.ds(i * _CHUNK, _CHUNK), :], bufs.at[i], in_sems.at[i]
        )

    def out_copy(i):
        return pltpu.make_async_copy(
            bufs.at[i], out_ref.at[pl.ds(i * _CHUNK, _CHUNK), :], out_sems.at[i]
        )

    for i in range(n_chunks):
        in_copy(i).start()
    for i in range(n_chunks):
        in_copy(i).wait()
        out_copy(i).start()
    for i in range(n_chunks):
        out_copy(i).wait()


def kernel(x, emb):
    n = x.shape[1]
    d = emb.shape[1]
    n_chunks = n // _CHUNK
    return pl.pallas_call(
        functools.partial(_copy_kernel, n, d),
        out_shape=jax.ShapeDtypeStruct((n, d), emb.dtype),
        in_specs=[pl.BlockSpec(memory_space=pl.ANY)],
        out_specs=pl.BlockSpec(memory_space=pl.ANY),
        scratch_shapes=[
            pltpu.VMEM((n_chunks, _CHUNK, d), emb.dtype),
            pltpu.SemaphoreType.DMA((n_chunks,)),
            pltpu.SemaphoreType.DMA((n_chunks,)),
        ],
    )(emb)
